# Initial kernel scaffold; baseline (speedup 1.0000x reference)
import jax
import jax.numpy as jnp
from jax.experimental import pallas as pl


def kernel(x, edge_index, pos_edge_index, neg_edge_index, W1, b1, W2, b2, Wc1, bc1, Wc2, bc2):
    raise NotImplementedError("write your pallas kernel here")


# speedup vs baseline: 9.1721x; 9.1721x over previous
import functools

import jax
import jax.numpy as jnp
from jax import lax
from jax.experimental import pallas as pl
from jax.experimental.pallas import tpu as pltpu
from jax.experimental.pallas import tpu_sc as plsc

N_NODES = 10000
N_PAD = 10240
D = 128
NC = 2
NS = 16
NW = NC * NS
ROWS_TILE = N_PAD // NS
ROWS_WORKER = N_PAD // NW
CH = 128

E_CONV = 320000 + N_NODES
CONV_CHUNKS = -(-E_CONV // (NW * CH))
E_CONV_PAD = CONV_CHUNKS * NW * CH
E_DEC = 320000
DEC_CHUNKS = -(-E_DEC // (NW * CH))
E_DEC_PAD = DEC_CHUNKS * NW * CH

_mesh = plsc.VectorSubcoreMesh(core_axis_name="c", subcore_axis_name="s")


def _worker_id():
    return lax.axis_index("s") * NC + lax.axis_index("c")


@functools.partial(
    pl.kernel,
    mesh=_mesh,
    compiler_params=pltpu.CompilerParams(needs_layout_passes=False),
    out_type=[
        jax.ShapeDtypeStruct((N_PAD,), jnp.float32),
        jax.ShapeDtypeStruct((N_PAD * D,), jnp.float32),
    ],
    scratch_types=[
        pltpu.VMEM_SHARED((N_PAD,), jnp.float32),
        pltpu.VMEM_SHARED((N_PAD,), jnp.float32),
        pltpu.VMEM((ROWS_TILE,), jnp.float32),
        pltpu.VMEM((CH,), jnp.float32),
        pltpu.VMEM((CH,), jnp.int32),
        pltpu.VMEM((ROWS_WORKER,), jnp.float32),
        pltpu.VMEM((ROWS_WORKER * D,), jnp.float32),
        pltpu.SemaphoreType.DMA,
    ],
)
def _deg_scale_kernel(dst_hbm, x_hbm, dinv_hbm, xs_hbm,
                      deg_sp, dinv_sp, rowbuf, ones_v, idx_v, dv, xv, sem):
    s = lax.axis_index("s")
    wid = _worker_id()
    tbase = s * ROWS_TILE

    for j in range(ROWS_TILE // 16):
        rowbuf[pl.ds(j * 16, 16)] = jnp.zeros((16,), jnp.float32)
    for j in range(CH // 16):
        ones_v[pl.ds(j * 16, 16)] = jnp.ones((16,), jnp.float32)
    pltpu.sync_copy(rowbuf, deg_sp.at[pl.ds(tbase, ROWS_TILE)])
    plsc.subcore_barrier()

    def deg_body(i, carry):
        base = (s * 2 * CONV_CHUNKS + i) * CH
        pltpu.sync_copy(dst_hbm.at[pl.ds(base, CH)], idx_v)
        pltpu.sync_copy(ones_v, deg_sp.at[idx_v], add=True)
        return carry

    lax.fori_loop(0, 2 * CONV_CHUNKS, deg_body, 0)
    plsc.subcore_barrier()

    pltpu.sync_copy(deg_sp.at[pl.ds(tbase, ROWS_TILE)], rowbuf)
    for j in range(ROWS_TILE // 16):
        d = rowbuf[pl.ds(j * 16, 16)]
        iy = jnp.int32(0x5F3759DF) - (lax.bitcast_convert_type(d, jnp.int32) >> 1)
        y = lax.bitcast_convert_type(iy, jnp.float32)
        for _ in range(3):
            y = y * (1.5 - 0.5 * d * y * y)
        rowbuf[pl.ds(j * 16, 16)] = y
    pltpu.sync_copy(rowbuf, dinv_sp.at[pl.ds(tbase, ROWS_TILE)])
    plsc.subcore_barrier()

    rbase = wid * ROWS_WORKER
    pltpu.sync_copy(dinv_sp.at[pl.ds(rbase, ROWS_WORKER)], dv)
    pltpu.sync_copy(dv, dinv_hbm.at[pl.ds(rbase, ROWS_WORKER)])
    pltpu.sync_copy(x_hbm.at[pl.ds(rbase * D, ROWS_WORKER * D)], xv)

    def scale_body(r, carry):
        bv = plsc.load_gather(dv, [jnp.full((16,), r, jnp.int32)])
        for c8 in range(D // 16):
            off = r * D + c8 * 16
            xv[pl.ds(off, 16)] = xv[pl.ds(off, 16)] * bv
        return carry

    lax.fori_loop(0, ROWS_WORKER, scale_body, 0)
    pltpu.sync_copy(xv, xs_hbm.at[pl.ds(rbase * D, ROWS_WORKER * D)])


@functools.partial(
    pl.kernel,
    mesh=_mesh,
    compiler_params=pltpu.CompilerParams(needs_layout_passes=False),
    out_type=jax.ShapeDtypeStruct((NC, N_PAD, D), jnp.float32),
    scratch_types=[
        pltpu.VMEM_SHARED((N_PAD, D), jnp.float32),
        pltpu.VMEM((32, D), jnp.float32),
        pltpu.VMEM((CH,), jnp.int32),
        pltpu.VMEM((CH,), jnp.int32),
        pltpu.VMEM((CH, D), jnp.float32),
        pltpu.SemaphoreType.DMA,
    ],
)
def _aggregate_kernel(src_hbm, dst_hbm, xs_hbm, out_hbm,
                      acc_sp, zbuf, idx_s, idx_d, rows, sem):
    c = lax.axis_index("c")
    s = lax.axis_index("s")
    wid = _worker_id()
    tbase = s * ROWS_TILE

    for i in range(32):
        for c8 in range(D // 16):
            zbuf[i, pl.ds(c8 * 16, 16)] = jnp.zeros((16,), jnp.float32)
    for k in range(ROWS_TILE // 32):
        pltpu.sync_copy(zbuf, acc_sp.at[pl.ds(tbase + k * 32, 32)])
    plsc.subcore_barrier()

    def edge_body(i, carry):
        base = (wid * CONV_CHUNKS + i) * CH
        pltpu.sync_copy(src_hbm.at[pl.ds(base, CH)], idx_s)
        pltpu.sync_copy(dst_hbm.at[pl.ds(base, CH)], idx_d)
        pltpu.async_copy(xs_hbm.at[idx_s], rows, sem).wait()
        pltpu.sync_copy(rows, acc_sp.at[idx_d], add=True)
        return carry

    lax.fori_loop(0, CONV_CHUNKS, edge_body, 0)
    plsc.subcore_barrier()

    pltpu.sync_copy(acc_sp.at[pl.ds(tbase, ROWS_TILE)],
                    out_hbm.at[c, pl.ds(tbase, ROWS_TILE)])


@functools.partial(
    pl.kernel,
    mesh=_mesh,
    compiler_params=pltpu.CompilerParams(needs_layout_passes=False),
    out_type=jax.ShapeDtypeStruct((E_DEC_PAD,), jnp.float32),
    scratch_types=[
        pltpu.VMEM((CH,), jnp.int32),
        pltpu.VMEM((CH,), jnp.int32),
        pltpu.VMEM((CH, D), jnp.float32),
        pltpu.VMEM((CH, D), jnp.float32),
        pltpu.VMEM((CH, 16), jnp.float32),
        pltpu.VMEM((CH,), jnp.float32),
        pltpu.VMEM((D,), jnp.float32),
        pltpu.VMEM((16,), jnp.float32),
        pltpu.SemaphoreType.DMA,
    ],
)
def _decode_kernel(src_hbm, dst_hbm, a_hbm, b_hbm, wc2_hbm, bc2_hbm, out_hbm,
                   idx_s, idx_d, a_rows, b_rows, pacc, outbuf, wcv, bcv, sem):
    wid = _worker_id()
    pltpu.sync_copy(wc2_hbm, wcv)
    pltpu.sync_copy(bc2_hbm, bcv)
    wch = [wcv[pl.ds(k * 16, 16)] for k in range(D // 16)]
    bc = bcv[...]
    ii = lax.iota(jnp.int32, 16)

    def chunk_body(i, carry):
        base = (wid * DEC_CHUNKS + i) * CH
        pltpu.sync_copy(src_hbm.at[pl.ds(base, CH)], idx_s)
        pltpu.sync_copy(dst_hbm.at[pl.ds(base, CH)], idx_d)
        pltpu.async_copy(a_hbm.at[idx_s], a_rows, sem).wait()
        pltpu.async_copy(b_hbm.at[idx_d], b_rows, sem).wait()

        def edge_body(e, carry2):
            er = jnp.full((16,), e, jnp.int32)
            acc = jnp.zeros((16,), jnp.float32)
            for c8 in range(D // 16):
                col = ii + (c8 * 16)
                av = plsc.load_gather(a_rows, [er, col])
                bv = plsc.load_gather(b_rows, [er, col])
                acc = acc + jnp.maximum(av + bv, 0.0) * wch[c8]
            plsc.store_scatter(pacc, [er, ii], acc)
            return carry2

        lax.fori_loop(0, CH, edge_body, 0)

        for g in range(CH // 16):
            tot = bc
            rowg = ii + (g * 16)
            for j in range(16):
                tot = tot + plsc.load_gather(pacc, [rowg, jnp.full((16,), j, jnp.int32)])
            outbuf[pl.ds(g * 16, 16)] = tot
        pltpu.sync_copy(outbuf, out_hbm.at[pl.ds(base, CH)])
        return carry

    lax.fori_loop(0, DEC_CHUNKS, chunk_body, 0)


_BLK = 512
_GRID = N_PAD // _BLK


def _tc1_body(p0, p1, dinv, w1, b1, out):
    dv = dinv[...]
    h = (p0[...] + p1[...]) * dv
    z = jnp.maximum(jnp.dot(h, w1[...], preferred_element_type=jnp.float32)
                    + b1[...], 0.0)
    out[...] = z * dv


def _tc1(p0, p1, dinv2d, W1, b1r):
    row_spec = pl.BlockSpec((_BLK, D), lambda i: (i, 0))
    return pl.pallas_call(
        _tc1_body,
        grid=(_GRID,),
        in_specs=[
            row_spec, row_spec,
            pl.BlockSpec((_BLK, 1), lambda i: (i, 0)),
            pl.BlockSpec((D, D), lambda i: (0, 0)),
            pl.BlockSpec((1, D), lambda i: (0, 0)),
        ],
        out_specs=row_spec,
        out_shape=jax.ShapeDtypeStruct((N_PAD, D), jnp.float32),
    )(p0, p1, dinv2d, W1, b1r)


def _tc2_body(q0, q1, dinv, w2, b2, wc1t, bc1, wc1b, a_out, b_out):
    dv = dinv[...]
    h = (q0[...] + q1[...]) * dv
    z2 = jnp.dot(h, w2[...], preferred_element_type=jnp.float32) + b2[...]
    a_out[...] = jnp.dot(z2, wc1t[...], preferred_element_type=jnp.float32) + bc1[...]
    b_out[...] = jnp.dot(z2, wc1b[...], preferred_element_type=jnp.float32)


def _tc2(q0, q1, dinv2d, W2, b2r, Wc1t, bc1r, Wc1b):
    row_spec = pl.BlockSpec((_BLK, D), lambda i: (i, 0))
    mat_spec = pl.BlockSpec((D, D), lambda i: (0, 0))
    vec_spec = pl.BlockSpec((1, D), lambda i: (0, 0))
    return pl.pallas_call(
        _tc2_body,
        grid=(_GRID,),
        in_specs=[row_spec, row_spec,
                  pl.BlockSpec((_BLK, 1), lambda i: (i, 0)),
                  mat_spec, vec_spec, mat_spec, vec_spec, mat_spec],
        out_specs=[row_spec, row_spec],
        out_shape=[jax.ShapeDtypeStruct((N_PAD, D), jnp.float32),
                   jax.ShapeDtypeStruct((N_PAD, D), jnp.float32)],
    )(q0, q1, dinv2d, W2, b2r, Wc1t, bc1r, Wc1b)


def kernel(x, edge_index, pos_edge_index, neg_edge_index,
           W1, b1, W2, b2, Wc1, bc1, Wc2, bc2):
    loop = jnp.arange(N_NODES, dtype=jnp.int32)
    src = jnp.concatenate([edge_index[0].astype(jnp.int32), loop,
                           jnp.arange(E_CONV_PAD - E_CONV, dtype=jnp.int32) % N_NODES])
    dst = jnp.concatenate([edge_index[1].astype(jnp.int32), loop,
                           N_NODES + jnp.arange(E_CONV_PAD - E_CONV, dtype=jnp.int32)
                           % (N_PAD - N_NODES)])

    x_pad = jnp.pad(x, ((0, N_PAD - N_NODES), (0, 0)))
    dinv, xs_flat = _deg_scale_kernel(dst, x_pad.reshape(-1))
    xs = xs_flat.reshape(N_PAD, D)
    dinv2d = dinv.reshape(N_PAD, 1)

    parts1 = _aggregate_kernel(src, dst, xs)
    z1s = _tc1(parts1[0], parts1[1], dinv2d, W1, b1.reshape(1, D))
    parts2 = _aggregate_kernel(src, dst, z1s)
    A, B = _tc2(parts2[0], parts2[1], dinv2d, W2, b2.reshape(1, D),
                Wc1[:D], bc1.reshape(1, D), Wc1[D:])

    dpad = jnp.arange(E_DEC_PAD - E_DEC, dtype=jnp.int32) % N_NODES
    dsrc = jnp.concatenate([pos_edge_index[0].astype(jnp.int32),
                            neg_edge_index[0].astype(jnp.int32), dpad])
    ddst = jnp.concatenate([pos_edge_index[1].astype(jnp.int32),
                            neg_edge_index[1].astype(jnp.int32), dpad])
    preds = _decode_kernel(dsrc, ddst, A, B, Wc2.reshape(D),
                           jnp.broadcast_to(bc2, (16,)).astype(jnp.float32))
    pos_pred = preds[:160000].reshape(160000, 1)
    neg_pred = preds[160000:320000].reshape(160000, 1)
    return (pos_pred, neg_pred)


# speedup vs baseline: 17.5243x; 1.9106x over previous
import functools

import jax
import jax.numpy as jnp
from jax import lax
from jax.experimental import pallas as pl
from jax.experimental.pallas import tpu as pltpu
from jax.experimental.pallas import tpu_sc as plsc

N_NODES = 10000
N_PAD = 10240
D = 128
NC = 2
NS = 16
NW = NC * NS
ROWS_TILE = N_PAD // NS
ROWS_WORKER = N_PAD // NW
CH = 128

E_CONV = 320000 + N_NODES
CONV_CHUNKS = 82
E_CONV_PAD = CONV_CHUNKS * NW * CH
DEG_CHUNKS = 2 * CONV_CHUNKS
E_DEC = 320000
DEC_CHUNKS = 80
E_DEC_PAD = DEC_CHUNKS * NW * CH

_mesh = plsc.VectorSubcoreMesh(core_axis_name="c", subcore_axis_name="s")
_params = pltpu.CompilerParams(needs_layout_passes=False)


def _worker_id():
    return lax.axis_index("s") * NC + lax.axis_index("c")


def _ch_slice(ref, i):
    return ref.at[pl.ds(pl.multiple_of(i * CH, CH), CH)]


@functools.partial(
    pl.kernel,
    mesh=_mesh,
    compiler_params=_params,
    out_type=[
        jax.ShapeDtypeStruct((N_PAD,), jnp.float32),
        jax.ShapeDtypeStruct((N_PAD * D,), jnp.float32),
    ],
    scratch_types=[
        pltpu.VMEM_SHARED((N_PAD,), jnp.float32),
        pltpu.VMEM_SHARED((N_PAD,), jnp.float32),
        pltpu.VMEM((CH,), jnp.int32),
        pltpu.VMEM((CH,), jnp.int32),
        pltpu.VMEM((CH,), jnp.int32),
        pltpu.VMEM((CH,), jnp.int32),
        pltpu.VMEM((CH,), jnp.int32),
        pltpu.VMEM((CH,), jnp.int32),
        pltpu.VMEM((CH,), jnp.int32),
        pltpu.VMEM((CH,), jnp.int32),
        pltpu.VMEM((ROWS_TILE,), jnp.float32),
        pltpu.VMEM((CH,), jnp.float32),
        pltpu.VMEM((ROWS_WORKER,), jnp.float32),
        pltpu.VMEM((ROWS_WORKER * D,), jnp.float32),
        pltpu.SemaphoreType.DMA,
        pltpu.SemaphoreType.DMA,
        pltpu.SemaphoreType.DMA,
    ],
)
def _deg_scale_kernel(dst_hbm, x_hbm, dinv_hbm, xs_hbm,
                      deg_sp, dinv_sp, i0, i1, i2, i3, i4, i5, i6, i7,
                      rowbuf, ones_v, dv, xv, isemA, isemB, ssem):
    s = lax.axis_index("s")
    wid = _worker_id()
    tbase = s * ROWS_TILE
    bufs = (i0, i1, i2, i3, i4, i5, i6, i7)
    isems = (isemA, isemB)
    ebase = s * DEG_CHUNKS * CH

    for j in range(ROWS_TILE // 16):
        rowbuf[pl.ds(j * 16, 16)] = jnp.zeros((16,), jnp.float32)
    for j in range(CH // 16):
        ones_v[pl.ds(j * 16, 16)] = jnp.ones((16,), jnp.float32)
    pltpu.sync_copy(rowbuf, deg_sp.at[pl.ds(tbase, ROWS_TILE)])
    plsc.subcore_barrier()

    NBLK = DEG_CHUNKS // 4
    for j in range(4):
        pltpu.async_copy(dst_hbm.at[pl.ds(ebase + j * CH, CH)], bufs[j], isemA)
    for blk in range(NBLK):
        g = blk % 2
        cur = bufs[g * 4:g * 4 + 4]
        nxt = bufs[(1 - g) * 4:(1 - g) * 4 + 4]
        if blk + 1 < NBLK:
            for j in range(4):
                off = ebase + ((blk + 1) * 4 + j) * CH
                pltpu.async_copy(dst_hbm.at[pl.ds(off, CH)], nxt[j],
                                 isems[1 - g])
        for j in range(4):
            off = ebase + (blk * 4 + j) * CH
            pltpu.make_async_copy(dst_hbm.at[pl.ds(off, CH)], cur[j],
                                  isems[g]).wait()
        descs = [pltpu.async_copy(ones_v, deg_sp.at[cur[j]], ssem, add=True)
                 for j in range(4)]
        for d_ in descs:
            d_.wait()
    plsc.subcore_barrier()

    pltpu.sync_copy(deg_sp.at[pl.ds(tbase, ROWS_TILE)], rowbuf)
    for j in range(ROWS_TILE // 16):
        d = rowbuf[pl.ds(j * 16, 16)]
        iy = jnp.int32(0x5F3759DF) - (lax.bitcast_convert_type(d, jnp.int32) >> 1)
        y = lax.bitcast_convert_type(iy, jnp.float32)
        for _ in range(3):
            y = y * (1.5 - 0.5 * d * y * y)
        rowbuf[pl.ds(j * 16, 16)] = y
    pltpu.sync_copy(rowbuf, dinv_sp.at[pl.ds(tbase, ROWS_TILE)])
    plsc.subcore_barrier()

    rbase = wid * ROWS_WORKER
    pltpu.sync_copy(dinv_sp.at[pl.ds(rbase, ROWS_WORKER)], dv)
    pltpu.sync_copy(dv, dinv_hbm.at[pl.ds(rbase, ROWS_WORKER)])
    pltpu.sync_copy(x_hbm.at[pl.ds(rbase * D, ROWS_WORKER * D)], xv)

    def scale_body(r, carry):
        bv = plsc.load_gather(dv, [jnp.full((16,), r, jnp.int32)])
        for c8 in range(D // 16):
            off = r * D + c8 * 16
            xv[pl.ds(off, 16)] = xv[pl.ds(off, 16)] * bv
        return carry

    lax.fori_loop(0, ROWS_WORKER, scale_body, 0, unroll=4)
    pltpu.sync_copy(xv, xs_hbm.at[pl.ds(rbase * D, ROWS_WORKER * D)])


@functools.partial(
    pl.kernel,
    mesh=_mesh,
    compiler_params=_params,
    out_type=jax.ShapeDtypeStruct((NC, N_PAD, D), jnp.float32),
    scratch_types=[
        pltpu.VMEM_SHARED((N_PAD, D), jnp.float32),
        pltpu.VMEM((CONV_CHUNKS * CH,), jnp.int32),
        pltpu.VMEM((CH,), jnp.int32),
        pltpu.VMEM((CH,), jnp.int32),
        pltpu.VMEM((32, D), jnp.float32),
        pltpu.VMEM((CH, D), jnp.float32),
        pltpu.VMEM((CH, D), jnp.float32),
        pltpu.SemaphoreType.DMA,
        pltpu.SemaphoreType.DMA,
        pltpu.SemaphoreType.DMA,
        pltpu.SemaphoreType.DMA,
    ],
)
def _aggregate_kernel(src_hbm, dst_hbm, xs_hbm, out_hbm,
                      acc_sp, srcv, idd0, idd1, zbuf, rows0, rows1,
                      isem0, isem1, gsem0, gsem1):
    c = lax.axis_index("c")
    s = lax.axis_index("s")
    wid = _worker_id()
    tbase = s * ROWS_TILE
    rows = (rows0, rows1)
    idd = (idd0, idd1)
    isem = (isem0, isem1)
    gsem = (gsem0, gsem1)
    ebase = wid * CONV_CHUNKS * CH

    spre = pltpu.async_copy(src_hbm.at[pl.ds(ebase, CONV_CHUNKS * CH)],
                            srcv, isem0)
    pltpu.async_copy(dst_hbm.at[pl.ds(ebase, CH)], idd0, isem0)

    for i in range(32):
        for c8 in range(D // 16):
            zbuf[i, pl.ds(c8 * 16, 16)] = jnp.zeros((16,), jnp.float32)
    zd = [pltpu.async_copy(zbuf, acc_sp.at[pl.ds(tbase + k * 32, 32)], gsem0)
          for k in range(ROWS_TILE // 32)]
    for d_ in zd:
        d_.wait()
    spre.wait()
    plsc.subcore_barrier()

    pltpu.async_copy(xs_hbm.at[_ch_slice(srcv, 0)], rows0, gsem0)

    def chunk_pair(g, carry):
        for b in range(2):
            i = 2 * g + b
            nb = 1 - b
            nx = jnp.where(i < CONV_CHUNKS - 1, i + 1, 0)
            pltpu.async_copy(dst_hbm.at[pl.ds(
                pl.multiple_of((ebase // CH + nx) * CH, CH), CH)],
                idd[nb], isem[nb])
            pltpu.make_async_copy(xs_hbm.at[_ch_slice(srcv, i)], rows[b],
                                  gsem[b]).wait()
            pltpu.async_copy(xs_hbm.at[_ch_slice(srcv, nx)], rows[nb],
                             gsem[nb])
            pltpu.make_async_copy(dst_hbm.at[pl.ds(ebase, CH)], idd[b],
                                  isem[b]).wait()
            pltpu.sync_copy(rows[b], acc_sp.at[idd[b]], add=True)
        return carry

    lax.fori_loop(0, CONV_CHUNKS // 2, chunk_pair, 0)
    pltpu.make_async_copy(xs_hbm.at[_ch_slice(srcv, 0)], rows0, gsem0).wait()
    pltpu.make_async_copy(dst_hbm.at[pl.ds(ebase, CH)], idd0, isem0).wait()
    plsc.subcore_barrier()

    pltpu.sync_copy(acc_sp.at[pl.ds(tbase, ROWS_TILE)],
                    out_hbm.at[c, pl.ds(tbase, ROWS_TILE)])


@functools.partial(
    pl.kernel,
    mesh=_mesh,
    compiler_params=_params,
    out_type=jax.ShapeDtypeStruct((E_DEC_PAD,), jnp.float32),
    scratch_types=[
        pltpu.VMEM((DEC_CHUNKS * CH,), jnp.int32),
        pltpu.VMEM((DEC_CHUNKS * CH,), jnp.int32),
        pltpu.VMEM((CH, D), jnp.float32),
        pltpu.VMEM((CH, D), jnp.float32),
        pltpu.VMEM((CH, D), jnp.float32),
        pltpu.VMEM((CH, D), jnp.float32),
        pltpu.VMEM((CH, 16), jnp.float32),
        pltpu.VMEM((CH,), jnp.float32),
        pltpu.VMEM((CH,), jnp.float32),
        pltpu.VMEM((D,), jnp.float32),
        pltpu.VMEM((16,), jnp.float32),
        pltpu.SemaphoreType.DMA,
        pltpu.SemaphoreType.DMA,
        pltpu.SemaphoreType.DMA,
        pltpu.SemaphoreType.DMA,
    ],
)
def _decode_kernel(src_hbm, dst_hbm, a_hbm, b_hbm, wc2_hbm, bc2_hbm, out_hbm,
                   srcv, dstv, ar0, ar1, br0, br1, pacc, ob0, ob1,
                   wcv, bcv, gsem0, gsem1, osem0, osem1):
    wid = _worker_id()
    ar = (ar0, ar1)
    br = (br0, br1)
    ob = (ob0, ob1)
    gsem = (gsem0, gsem1)
    osem = (osem0, osem1)

    ebase = wid * DEC_CHUNKS * CH
    pltpu.sync_copy(src_hbm.at[pl.ds(ebase, DEC_CHUNKS * CH)], srcv)
    pltpu.sync_copy(dst_hbm.at[pl.ds(ebase, DEC_CHUNKS * CH)], dstv)
    pltpu.sync_copy(wc2_hbm, wcv)
    pltpu.sync_copy(bc2_hbm, bcv)
    wch = [wcv[pl.ds(k * 16, 16)] for k in range(D // 16)]
    bc = bcv[...]
    ii = lax.iota(jnp.int32, 16)

    pltpu.async_copy(a_hbm.at[_ch_slice(srcv, 0)], ar0, gsem0)
    pltpu.async_copy(b_hbm.at[_ch_slice(dstv, 0)], br0, gsem0)

    def chunk_pair(g, carry):
        for b in range(2):
            i = 2 * g + b
            nb = 1 - b
            nx = jnp.where(i < DEC_CHUNKS - 1, i + 1, 0)
            pltpu.make_async_copy(a_hbm.at[_ch_slice(srcv, i)], ar[b],
                                  gsem[b]).wait()
            pltpu.make_async_copy(b_hbm.at[_ch_slice(dstv, i)], br[b],
                                  gsem[b]).wait()
            pltpu.async_copy(a_hbm.at[_ch_slice(srcv, nx)], ar[nb], gsem[nb])
            pltpu.async_copy(b_hbm.at[_ch_slice(dstv, nx)], br[nb], gsem[nb])

            a_r, b_r = ar[b], br[b]

            def edge_body(e, carry2):
                er = jnp.full((16,), e, jnp.int32)
                acc = jnp.zeros((16,), jnp.float32)
                for c8 in range(D // 16):
                    col = ii + (c8 * 16)
                    av = plsc.load_gather(a_r, [er, col])
                    bv = plsc.load_gather(b_r, [er, col])
                    acc = acc + jnp.maximum(av + bv, 0.0) * wch[c8]
                plsc.store_scatter(pacc, [er, ii], acc)
                return carry2

            lax.fori_loop(0, CH, edge_body, 0, unroll=8)

            @pl.when(i >= 2)
            def _():
                pltpu.make_async_copy(
                    ob[b], out_hbm.at[pl.ds(ebase, CH)], osem[b]).wait()

            for gg in range(CH // 16):
                tot = bc
                rowg = ii + (gg * 16)
                for j in range(16):
                    tot = tot + plsc.load_gather(
                        pacc, [rowg, jnp.full((16,), j, jnp.int32)])
                ob[b][pl.ds(gg * 16, 16)] = tot
            pltpu.async_copy(
                ob[b],
                out_hbm.at[pl.ds(pl.multiple_of((ebase // CH + i) * CH, CH),
                                 CH)],
                osem[b])
        return carry

    lax.fori_loop(0, DEC_CHUNKS // 2, chunk_pair, 0)
    pltpu.make_async_copy(a_hbm.at[_ch_slice(srcv, 0)], ar0, gsem0).wait()
    pltpu.make_async_copy(b_hbm.at[_ch_slice(dstv, 0)], br0, gsem0).wait()
    pltpu.make_async_copy(ob0, out_hbm.at[pl.ds(ebase, CH)], osem0).wait()
    pltpu.make_async_copy(ob1, out_hbm.at[pl.ds(ebase, CH)], osem1).wait()


_BLK = 512
_GRID = N_PAD // _BLK


def _tc1_body(p0, p1, dinv, w1, b1, out):
    dv = dinv[...]
    h = (p0[...] + p1[...]) * dv
    z = jnp.maximum(jnp.dot(h, w1[...], preferred_element_type=jnp.float32)
                    + b1[...], 0.0)
    out[...] = z * dv


def _tc1(p0, p1, dinv2d, W1, b1r):
    row_spec = pl.BlockSpec((_BLK, D), lambda i: (i, 0))
    return pl.pallas_call(
        _tc1_body,
        grid=(_GRID,),
        in_specs=[
            row_spec, row_spec,
            pl.BlockSpec((_BLK, 1), lambda i: (i, 0)),
            pl.BlockSpec((D, D), lambda i: (0, 0)),
            pl.BlockSpec((1, D), lambda i: (0, 0)),
        ],
        out_specs=row_spec,
        out_shape=jax.ShapeDtypeStruct((N_PAD, D), jnp.float32),
    )(p0, p1, dinv2d, W1, b1r)


def _tc2_body(q0, q1, dinv, w2, b2, wc1t, bc1, wc1b, a_out, b_out):
    dv = dinv[...]
    h = (q0[...] + q1[...]) * dv
    z2 = jnp.dot(h, w2[...], preferred_element_type=jnp.float32) + b2[...]
    a_out[...] = jnp.dot(z2, wc1t[...], preferred_element_type=jnp.float32) + bc1[...]
    b_out[...] = jnp.dot(z2, wc1b[...], preferred_element_type=jnp.float32)


def _tc2(q0, q1, dinv2d, W2, b2r, Wc1t, bc1r, Wc1b):
    row_spec = pl.BlockSpec((_BLK, D), lambda i: (i, 0))
    mat_spec = pl.BlockSpec((D, D), lambda i: (0, 0))
    vec_spec = pl.BlockSpec((1, D), lambda i: (0, 0))
    return pl.pallas_call(
        _tc2_body,
        grid=(_GRID,),
        in_specs=[row_spec, row_spec,
                  pl.BlockSpec((_BLK, 1), lambda i: (i, 0)),
                  mat_spec, vec_spec, mat_spec, vec_spec, mat_spec],
        out_specs=[row_spec, row_spec],
        out_shape=[jax.ShapeDtypeStruct((N_PAD, D), jnp.float32),
                   jax.ShapeDtypeStruct((N_PAD, D), jnp.float32)],
    )(q0, q1, dinv2d, W2, b2r, Wc1t, bc1r, Wc1b)


def kernel(x, edge_index, pos_edge_index, neg_edge_index,
           W1, b1, W2, b2, Wc1, bc1, Wc2, bc2):
    loop = jnp.arange(N_NODES, dtype=jnp.int32)
    src = jnp.concatenate([edge_index[0].astype(jnp.int32), loop,
                           jnp.arange(E_CONV_PAD - E_CONV, dtype=jnp.int32) % N_NODES])
    dst = jnp.concatenate([edge_index[1].astype(jnp.int32), loop,
                           N_NODES + jnp.arange(E_CONV_PAD - E_CONV, dtype=jnp.int32)
                           % (N_PAD - N_NODES)])

    x_pad = jnp.pad(x, ((0, N_PAD - N_NODES), (0, 0)))
    dinv, xs_flat = _deg_scale_kernel(dst, x_pad.reshape(-1))
    xs = xs_flat.reshape(N_PAD, D)
    dinv2d = dinv.reshape(N_PAD, 1)

    parts1 = _aggregate_kernel(src, dst, xs)
    z1s = _tc1(parts1[0], parts1[1], dinv2d, W1, b1.reshape(1, D))
    parts2 = _aggregate_kernel(src, dst, z1s)
    A, B = _tc2(parts2[0], parts2[1], dinv2d, W2, b2.reshape(1, D),
                Wc1[:D], bc1.reshape(1, D), Wc1[D:])

    dpad = jnp.arange(E_DEC_PAD - E_DEC, dtype=jnp.int32) % N_NODES
    dsrc = jnp.concatenate([pos_edge_index[0].astype(jnp.int32),
                            neg_edge_index[0].astype(jnp.int32), dpad])
    ddst = jnp.concatenate([pos_edge_index[1].astype(jnp.int32),
                            neg_edge_index[1].astype(jnp.int32), dpad])
    preds = _decode_kernel(dsrc, ddst, A, B, Wc2.reshape(D),
                           jnp.broadcast_to(bc2, (16,)).astype(jnp.float32))
    pos_pred = preds[:160000].reshape(160000, 1)
    neg_pred = preds[160000:320000].reshape(160000, 1)
    return (pos_pred, neg_pred)


# speedup vs baseline: 17.6193x; 1.0054x over previous
import functools

import jax
import jax.numpy as jnp
from jax import lax
from jax.experimental import pallas as pl
from jax.experimental.pallas import tpu as pltpu
from jax.experimental.pallas import tpu_sc as plsc

N_NODES = 10000
N_PAD = 10240
D = 128
NC = 2
NS = 16
NW = NC * NS
ROWS_TILE = N_PAD // NS
ROWS_WORKER = N_PAD // NW
CH = 128

E_CONV = 320000 + N_NODES
CONV_CHUNKS = 82
E_CONV_PAD = CONV_CHUNKS * NW * CH
DEG_CHUNKS = 2 * CONV_CHUNKS
E_DEC = 320000
DEC_CHUNKS = 80
E_DEC_PAD = DEC_CHUNKS * NW * CH

_mesh = plsc.VectorSubcoreMesh(core_axis_name="c", subcore_axis_name="s")
_params = pltpu.CompilerParams(needs_layout_passes=False)


def _worker_id():
    return lax.axis_index("s") * NC + lax.axis_index("c")


def _ch_slice(ref, i):
    return ref.at[pl.ds(pl.multiple_of(i * CH, CH), CH)]


@functools.partial(
    pl.kernel,
    mesh=_mesh,
    compiler_params=_params,
    out_type=[
        jax.ShapeDtypeStruct((N_PAD,), jnp.float32),
        jax.ShapeDtypeStruct((N_PAD * D,), jnp.float32),
    ],
    scratch_types=[
        pltpu.VMEM_SHARED((N_PAD,), jnp.float32),
        pltpu.VMEM_SHARED((N_PAD,), jnp.float32),
        pltpu.VMEM((CH,), jnp.int32),
        pltpu.VMEM((CH,), jnp.int32),
        pltpu.VMEM((CH,), jnp.int32),
        pltpu.VMEM((CH,), jnp.int32),
        pltpu.VMEM((CH,), jnp.int32),
        pltpu.VMEM((CH,), jnp.int32),
        pltpu.VMEM((CH,), jnp.int32),
        pltpu.VMEM((CH,), jnp.int32),
        pltpu.VMEM((ROWS_TILE,), jnp.float32),
        pltpu.VMEM((CH,), jnp.float32),
        pltpu.VMEM((ROWS_WORKER,), jnp.float32),
        pltpu.VMEM((ROWS_WORKER * D,), jnp.float32),
        pltpu.SemaphoreType.DMA,
        pltpu.SemaphoreType.DMA,
        pltpu.SemaphoreType.DMA,
    ],
)
def _deg_scale_kernel(dst_hbm, x_hbm, dinv_hbm, xs_hbm,
                      deg_sp, dinv_sp, i0, i1, i2, i3, i4, i5, i6, i7,
                      rowbuf, ones_v, dv, xv, isemA, isemB, ssem):
    s = lax.axis_index("s")
    wid = _worker_id()
    tbase = s * ROWS_TILE
    bufs = (i0, i1, i2, i3, i4, i5, i6, i7)
    isems = (isemA, isemB)
    ebase = s * DEG_CHUNKS * CH

    for j in range(ROWS_TILE // 16):
        rowbuf[pl.ds(j * 16, 16)] = jnp.zeros((16,), jnp.float32)
    for j in range(CH // 16):
        ones_v[pl.ds(j * 16, 16)] = jnp.ones((16,), jnp.float32)
    pltpu.sync_copy(rowbuf, deg_sp.at[pl.ds(tbase, ROWS_TILE)])
    plsc.subcore_barrier()

    NBLK = DEG_CHUNKS // 4
    for j in range(4):
        pltpu.async_copy(dst_hbm.at[pl.ds(ebase + j * CH, CH)], bufs[j], isemA)
    for blk in range(NBLK):
        g = blk % 2
        cur = bufs[g * 4:g * 4 + 4]
        nxt = bufs[(1 - g) * 4:(1 - g) * 4 + 4]
        if blk + 1 < NBLK:
            for j in range(4):
                off = ebase + ((blk + 1) * 4 + j) * CH
                pltpu.async_copy(dst_hbm.at[pl.ds(off, CH)], nxt[j],
                                 isems[1 - g])
        for j in range(4):
            off = ebase + (blk * 4 + j) * CH
            pltpu.make_async_copy(dst_hbm.at[pl.ds(off, CH)], cur[j],
                                  isems[g]).wait()
        descs = [pltpu.async_copy(ones_v, deg_sp.at[cur[j]], ssem, add=True)
                 for j in range(4)]
        for d_ in descs:
            d_.wait()
    plsc.subcore_barrier()

    pltpu.sync_copy(deg_sp.at[pl.ds(tbase, ROWS_TILE)], rowbuf)
    for j in range(ROWS_TILE // 16):
        d = rowbuf[pl.ds(j * 16, 16)]
        iy = jnp.int32(0x5F3759DF) - (lax.bitcast_convert_type(d, jnp.int32) >> 1)
        y = lax.bitcast_convert_type(iy, jnp.float32)
        for _ in range(3):
            y = y * (1.5 - 0.5 * d * y * y)
        rowbuf[pl.ds(j * 16, 16)] = y
    pltpu.sync_copy(rowbuf, dinv_sp.at[pl.ds(tbase, ROWS_TILE)])
    plsc.subcore_barrier()

    rbase = wid * ROWS_WORKER
    pltpu.sync_copy(dinv_sp.at[pl.ds(rbase, ROWS_WORKER)], dv)
    pltpu.sync_copy(dv, dinv_hbm.at[pl.ds(rbase, ROWS_WORKER)])
    pltpu.sync_copy(x_hbm.at[pl.ds(rbase * D, ROWS_WORKER * D)], xv)

    def scale_body(r, carry):
        bv = plsc.load_gather(dv, [jnp.full((16,), r, jnp.int32)])
        for c8 in range(D // 16):
            off = r * D + c8 * 16
            xv[pl.ds(off, 16)] = xv[pl.ds(off, 16)] * bv
        return carry

    lax.fori_loop(0, ROWS_WORKER, scale_body, 0, unroll=4)
    pltpu.sync_copy(xv, xs_hbm.at[pl.ds(rbase * D, ROWS_WORKER * D)])


@functools.partial(
    pl.kernel,
    mesh=_mesh,
    compiler_params=_params,
    out_type=jax.ShapeDtypeStruct((NC, N_PAD, D), jnp.float32),
    scratch_types=[
        pltpu.VMEM_SHARED((N_PAD, D), jnp.float32),
        pltpu.VMEM((CONV_CHUNKS * CH,), jnp.int32),
        pltpu.VMEM((CH,), jnp.int32),
        pltpu.VMEM((CH,), jnp.int32),
        pltpu.VMEM((32, D), jnp.float32),
        pltpu.VMEM((CH, D), jnp.float32),
        pltpu.VMEM((CH, D), jnp.float32),
        pltpu.SemaphoreType.DMA,
        pltpu.SemaphoreType.DMA,
        pltpu.SemaphoreType.DMA,
        pltpu.SemaphoreType.DMA,
    ],
)
def _aggregate_kernel(src_hbm, dst_hbm, xs_hbm, out_hbm,
                      acc_sp, srcv, idd0, idd1, zbuf, rows0, rows1,
                      isem0, isem1, gsem0, gsem1):
    c = lax.axis_index("c")
    s = lax.axis_index("s")
    wid = _worker_id()
    tbase = s * ROWS_TILE
    rows = (rows0, rows1)
    idd = (idd0, idd1)
    isem = (isem0, isem1)
    gsem = (gsem0, gsem1)
    ebase = wid * CONV_CHUNKS * CH

    spre = pltpu.async_copy(src_hbm.at[pl.ds(ebase, CONV_CHUNKS * CH)],
                            srcv, isem0)
    pltpu.async_copy(dst_hbm.at[pl.ds(ebase, CH)], idd0, isem0)

    for i in range(32):
        for c8 in range(D // 16):
            zbuf[i, pl.ds(c8 * 16, 16)] = jnp.zeros((16,), jnp.float32)
    zd = [pltpu.async_copy(zbuf, acc_sp.at[pl.ds(tbase + k * 32, 32)], gsem0)
          for k in range(ROWS_TILE // 32)]
    for d_ in zd:
        d_.wait()
    spre.wait()
    plsc.subcore_barrier()

    pltpu.async_copy(xs_hbm.at[_ch_slice(srcv, 0)], rows0, gsem0)

    def chunk_pair(g, carry):
        for b in range(2):
            i = 2 * g + b
            nb = 1 - b
            nx = jnp.where(i < CONV_CHUNKS - 1, i + 1, 0)
            pltpu.async_copy(dst_hbm.at[pl.ds(
                pl.multiple_of((ebase // CH + nx) * CH, CH), CH)],
                idd[nb], isem[nb])
            pltpu.make_async_copy(xs_hbm.at[_ch_slice(srcv, i)], rows[b],
                                  gsem[b]).wait()
            pltpu.async_copy(xs_hbm.at[_ch_slice(srcv, nx)], rows[nb],
                             gsem[nb])
            pltpu.make_async_copy(dst_hbm.at[pl.ds(ebase, CH)], idd[b],
                                  isem[b]).wait()
            pltpu.sync_copy(rows[b], acc_sp.at[idd[b]], add=True)
        return carry

    lax.fori_loop(0, CONV_CHUNKS // 2, chunk_pair, 0)
    pltpu.make_async_copy(xs_hbm.at[_ch_slice(srcv, 0)], rows0, gsem0).wait()
    pltpu.make_async_copy(dst_hbm.at[pl.ds(ebase, CH)], idd0, isem0).wait()
    plsc.subcore_barrier()

    pltpu.sync_copy(acc_sp.at[pl.ds(tbase, ROWS_TILE)],
                    out_hbm.at[c, pl.ds(tbase, ROWS_TILE)])


@functools.partial(
    pl.kernel,
    mesh=_mesh,
    compiler_params=_params,
    out_type=jax.ShapeDtypeStruct((E_DEC_PAD,), jnp.float32),
    scratch_types=[
        pltpu.VMEM((DEC_CHUNKS * CH,), jnp.int32),
        pltpu.VMEM((DEC_CHUNKS * CH,), jnp.int32),
        pltpu.VMEM((CH, D), jnp.float32),
        pltpu.VMEM((CH, D), jnp.float32),
        pltpu.VMEM((CH, D), jnp.float32),
        pltpu.VMEM((CH, D), jnp.float32),
        pltpu.VMEM((CH, 16), jnp.float32),
        pltpu.VMEM((CH,), jnp.float32),
        pltpu.VMEM((CH,), jnp.float32),
        pltpu.VMEM((D,), jnp.float32),
        pltpu.VMEM((16,), jnp.float32),
        pltpu.SemaphoreType.DMA,
        pltpu.SemaphoreType.DMA,
        pltpu.SemaphoreType.DMA,
        pltpu.SemaphoreType.DMA,
    ],
)
def _decode_kernel(src_hbm, dst_hbm, a_hbm, b_hbm, wc2_hbm, bc2_hbm, out_hbm,
                   srcv, dstv, ar0, ar1, br0, br1, pacc, ob0, ob1,
                   wcv, bcv, gsem0, gsem1, osem0, osem1):
    wid = _worker_id()
    ar = (ar0, ar1)
    br = (br0, br1)
    ob = (ob0, ob1)
    gsem = (gsem0, gsem1)
    osem = (osem0, osem1)

    ebase = wid * DEC_CHUNKS * CH
    pltpu.sync_copy(src_hbm.at[pl.ds(ebase, DEC_CHUNKS * CH)], srcv)
    pltpu.sync_copy(dst_hbm.at[pl.ds(ebase, DEC_CHUNKS * CH)], dstv)
    pltpu.sync_copy(wc2_hbm, wcv)
    pltpu.sync_copy(bc2_hbm, bcv)
    wch = [wcv[pl.ds(k * 16, 16)] for k in range(D // 16)]
    bc = bcv[...]
    ii = lax.iota(jnp.int32, 16)

    pltpu.async_copy(a_hbm.at[_ch_slice(srcv, 0)], ar0, gsem0)
    pltpu.async_copy(b_hbm.at[_ch_slice(dstv, 0)], br0, gsem0)

    def chunk_pair(g, carry):
        for b in range(2):
            i = 2 * g + b
            nb = 1 - b
            nx = jnp.where(i < DEC_CHUNKS - 1, i + 1, 0)
            pltpu.make_async_copy(a_hbm.at[_ch_slice(srcv, i)], ar[b],
                                  gsem[b]).wait()
            pltpu.make_async_copy(b_hbm.at[_ch_slice(dstv, i)], br[b],
                                  gsem[b]).wait()
            pltpu.async_copy(a_hbm.at[_ch_slice(srcv, nx)], ar[nb], gsem[nb])
            pltpu.async_copy(b_hbm.at[_ch_slice(dstv, nx)], br[nb], gsem[nb])

            a_r, b_r = ar[b], br[b]

            def edge_body(e, carry2):
                acc = jnp.zeros((16,), jnp.float32)
                for c8 in range(D // 16):
                    av = a_r[e, pl.ds(c8 * 16, 16)]
                    bv = b_r[e, pl.ds(c8 * 16, 16)]
                    acc = acc + jnp.maximum(av + bv, 0.0) * wch[c8]
                pacc[e, pl.ds(0, 16)] = acc
                return carry2

            lax.fori_loop(0, CH, edge_body, 0, unroll=8)

            @pl.when(i >= 2)
            def _():
                pltpu.make_async_copy(
                    ob[b], out_hbm.at[pl.ds(ebase, CH)], osem[b]).wait()

            for gg in range(CH // 16):
                tot = bc
                rowg = ii + (gg * 16)
                for j in range(16):
                    tot = tot + plsc.load_gather(
                        pacc, [rowg, jnp.full((16,), j, jnp.int32)])
                ob[b][pl.ds(gg * 16, 16)] = tot
            pltpu.async_copy(
                ob[b],
                out_hbm.at[pl.ds(pl.multiple_of((ebase // CH + i) * CH, CH),
                                 CH)],
                osem[b])
        return carry

    lax.fori_loop(0, DEC_CHUNKS // 2, chunk_pair, 0)
    pltpu.make_async_copy(a_hbm.at[_ch_slice(srcv, 0)], ar0, gsem0).wait()
    pltpu.make_async_copy(b_hbm.at[_ch_slice(dstv, 0)], br0, gsem0).wait()
    pltpu.make_async_copy(ob0, out_hbm.at[pl.ds(ebase, CH)], osem0).wait()
    pltpu.make_async_copy(ob1, out_hbm.at[pl.ds(ebase, CH)], osem1).wait()


_BLK = 512
_GRID = N_PAD // _BLK


def _tc1_body(p0, p1, dinv, w1, b1, out):
    dv = dinv[...]
    h = (p0[...] + p1[...]) * dv
    z = jnp.maximum(jnp.dot(h, w1[...], preferred_element_type=jnp.float32)
                    + b1[...], 0.0)
    out[...] = z * dv


def _tc1(p0, p1, dinv2d, W1, b1r):
    row_spec = pl.BlockSpec((_BLK, D), lambda i: (i, 0))
    return pl.pallas_call(
        _tc1_body,
        grid=(_GRID,),
        in_specs=[
            row_spec, row_spec,
            pl.BlockSpec((_BLK, 1), lambda i: (i, 0)),
            pl.BlockSpec((D, D), lambda i: (0, 0)),
            pl.BlockSpec((1, D), lambda i: (0, 0)),
        ],
        out_specs=row_spec,
        out_shape=jax.ShapeDtypeStruct((N_PAD, D), jnp.float32),
    )(p0, p1, dinv2d, W1, b1r)


def _tc2_body(q0, q1, dinv, w2, b2, wc1t, bc1, wc1b, a_out, b_out):
    dv = dinv[...]
    h = (q0[...] + q1[...]) * dv
    z2 = jnp.dot(h, w2[...], preferred_element_type=jnp.float32) + b2[...]
    a_out[...] = jnp.dot(z2, wc1t[...], preferred_element_type=jnp.float32) + bc1[...]
    b_out[...] = jnp.dot(z2, wc1b[...], preferred_element_type=jnp.float32)


def _tc2(q0, q1, dinv2d, W2, b2r, Wc1t, bc1r, Wc1b):
    row_spec = pl.BlockSpec((_BLK, D), lambda i: (i, 0))
    mat_spec = pl.BlockSpec((D, D), lambda i: (0, 0))
    vec_spec = pl.BlockSpec((1, D), lambda i: (0, 0))
    return pl.pallas_call(
        _tc2_body,
        grid=(_GRID,),
        in_specs=[row_spec, row_spec,
                  pl.BlockSpec((_BLK, 1), lambda i: (i, 0)),
                  mat_spec, vec_spec, mat_spec, vec_spec, mat_spec],
        out_specs=[row_spec, row_spec],
        out_shape=[jax.ShapeDtypeStruct((N_PAD, D), jnp.float32),
                   jax.ShapeDtypeStruct((N_PAD, D), jnp.float32)],
    )(q0, q1, dinv2d, W2, b2r, Wc1t, bc1r, Wc1b)


def kernel(x, edge_index, pos_edge_index, neg_edge_index,
           W1, b1, W2, b2, Wc1, bc1, Wc2, bc2):
    loop = jnp.arange(N_NODES, dtype=jnp.int32)
    src = jnp.concatenate([edge_index[0].astype(jnp.int32), loop,
                           jnp.arange(E_CONV_PAD - E_CONV, dtype=jnp.int32) % N_NODES])
    dst = jnp.concatenate([edge_index[1].astype(jnp.int32), loop,
                           N_NODES + jnp.arange(E_CONV_PAD - E_CONV, dtype=jnp.int32)
                           % (N_PAD - N_NODES)])

    x_pad = jnp.pad(x, ((0, N_PAD - N_NODES), (0, 0)))
    dinv, xs_flat = _deg_scale_kernel(dst, x_pad.reshape(-1))
    xs = xs_flat.reshape(N_PAD, D)
    dinv2d = dinv.reshape(N_PAD, 1)

    parts1 = _aggregate_kernel(src, dst, xs)
    z1s = _tc1(parts1[0], parts1[1], dinv2d, W1, b1.reshape(1, D))
    parts2 = _aggregate_kernel(src, dst, z1s)
    A, B = _tc2(parts2[0], parts2[1], dinv2d, W2, b2.reshape(1, D),
                Wc1[:D], bc1.reshape(1, D), Wc1[D:])

    dpad = jnp.arange(E_DEC_PAD - E_DEC, dtype=jnp.int32) % N_NODES
    dsrc = jnp.concatenate([pos_edge_index[0].astype(jnp.int32),
                            neg_edge_index[0].astype(jnp.int32), dpad])
    ddst = jnp.concatenate([pos_edge_index[1].astype(jnp.int32),
                            neg_edge_index[1].astype(jnp.int32), dpad])
    preds = _decode_kernel(dsrc, ddst, A, B, Wc2.reshape(D),
                           jnp.broadcast_to(bc2, (16,)).astype(jnp.float32))
    pos_pred = preds[:160000].reshape(160000, 1)
    neg_pred = preds[160000:320000].reshape(160000, 1)
    return (pos_pred, neg_pred)
